# baseline (speedup 1.0000x reference)
import functools
import jax
import jax.numpy as jnp
from jax import lax
from jax.experimental import pallas as pl
from jax.experimental.pallas import tpu as pltpu
from jax.experimental.pallas import tpu_sc as plsc

NUM_BINS = 128
EMB_DIM = 64
NC = 2
NS = 16
NW = NC * NS

CHUNK = 128
KFIRE = 8


def _idx_body(num_ref, idx_ref):
    x = num_ref[...]
    l = jnp.log2(jnp.abs(x) + 1e-10)
    t = (l + 10.0) * 6.4
    k = jnp.clip(jnp.ceil(t).astype(jnp.int32), 0, 129)
    bk = -10.0 + k.astype(jnp.float32) * 0.15625
    bkm1 = -10.0 + (k - 1).astype(jnp.float32) * 0.15625
    k = jnp.where((k <= 128) & (bk < l), k + 1, k)
    k = jnp.where((k >= 1) & (bkm1 >= l), k - 1, k)
    b = jnp.clip(k, 0, 127)
    sgn = (x > 0.0).astype(jnp.int32) - (x < 0.0).astype(jnp.int32)
    idx_ref[...] = b + 128 * (sgn + 1)


def _table_body(w_ref, s_ref, out_ref):
    ws = w_ref[...] * s_ref[...]
    out_ref[0:NUM_BINS, :] = -ws
    out_ref[NUM_BINS:2 * NUM_BINS, :] = jnp.zeros_like(ws)
    out_ref[2 * NUM_BINS:3 * NUM_BINS, :] = ws


def _sc_gather(idx_hbm, w3_hbm, out_hbm, idx_v, buf_v, gsem, wsem):
    wid = lax.axis_index("s") * NC + lax.axis_index("c")
    chunks_per_w = idx_hbm.shape[1]
    base = wid * chunks_per_w * CHUNK

    pltpu.sync_copy(idx_hbm.at[wid], idx_v)

    ngroups = chunks_per_w // KFIRE

    def out_slice(g):
        return out_hbm.at[pl.ds(base + g * CHUNK, CHUNK)]

    for j in range(KFIRE):
        pltpu.async_copy(buf_v.at[j], out_slice(j), wsem)

    def group(g0, carry):
        gathers = []
        for j in range(KFIRE):
            g = g0 * KFIRE + j
            pltpu.make_async_copy(buf_v.at[j], out_slice(g), wsem).wait()
            gathers.append(
                pltpu.async_copy(w3_hbm.at[idx_v.at[g]], buf_v.at[j], gsem)
            )
        for j in range(KFIRE):
            g = g0 * KFIRE + j
            gathers[j].wait()
            pltpu.async_copy(buf_v.at[j], out_slice(g), wsem)
        return carry

    lax.fori_loop(0, ngroups, group, 0)

    for j in range(KFIRE):
        pltpu.make_async_copy(buf_v.at[j], out_slice(j), wsem).wait()


def kernel(number, W, scale):
    squeeze = number.ndim == 1
    if squeeze:
        number = number[None, :]
    B, N = number.shape
    M = B * N
    assert M % (NW * CHUNK) == 0
    chunks_per_w = M // (NW * CHUNK)

    rows_blk = max(8, min(B, (1 << 22) // (4 * N)))
    while B % rows_blk:
        rows_blk //= 2
    idx = pl.pallas_call(
        _idx_body,
        grid=(B // rows_blk,),
        in_specs=[pl.BlockSpec((rows_blk, N), lambda i: (i, 0))],
        out_specs=pl.BlockSpec((rows_blk, N), lambda i: (i, 0)),
        out_shape=jax.ShapeDtypeStruct((B, N), jnp.int32),
    )(number)

    w3 = pl.pallas_call(
        _table_body,
        out_shape=jax.ShapeDtypeStruct((3 * NUM_BINS, EMB_DIM), jnp.float32),
    )(W, scale.reshape(NUM_BINS, 1))

    idx3 = idx.reshape(NW, chunks_per_w, CHUNK)

    mesh = plsc.VectorSubcoreMesh(
        core_axis_name="c", subcore_axis_name="s", num_cores=NC, num_subcores=NS
    )
    out = pl.kernel(
        _sc_gather,
        out_type=jax.ShapeDtypeStruct((M, EMB_DIM), jnp.float32),
        mesh=mesh,
        scratch_types=[
            pltpu.VMEM((chunks_per_w, CHUNK), jnp.int32),
            pltpu.VMEM((KFIRE, CHUNK, EMB_DIM), jnp.float32),
            pltpu.SemaphoreType.DMA,
            pltpu.SemaphoreType.DMA,
        ],
        compiler_params=pltpu.CompilerParams(use_tc_tiling_on_sc=False),
    )(idx3, w3)

    out = out.reshape(B, N, EMB_DIM)
    if squeeze:
        out = out[0]
    return out


# speedup vs baseline: 3.6169x; 3.6169x over previous
import functools
import jax
import jax.numpy as jnp
from jax import lax
from jax.experimental import pallas as pl
from jax.experimental.pallas import tpu as pltpu
from jax.experimental.pallas import tpu_sc as plsc

NUM_BINS = 128
EMB_DIM = 64
NC = 2
NS = 16
NW = NC * NS

CHUNK = 128
KFIRE = 8


def _idx_body(num_ref, idx_ref):
    x = num_ref[...]
    l = jnp.log2(jnp.abs(x) + 1e-10)
    t = (l + 10.0) * 6.4
    k = jnp.clip(jnp.ceil(t).astype(jnp.int32), 0, 129)
    bk = -10.0 + k.astype(jnp.float32) * 0.15625
    bkm1 = -10.0 + (k - 1).astype(jnp.float32) * 0.15625
    k = jnp.where((k <= 128) & (bk < l), k + 1, k)
    k = jnp.where((k >= 1) & (bkm1 >= l), k - 1, k)
    b = jnp.clip(k, 0, 127)
    sgn = (x > 0.0).astype(jnp.int32) - (x < 0.0).astype(jnp.int32)
    idx_ref[...] = b + 128 * (sgn + 1)


def _table_body(w_ref, s_ref, out_ref):
    ws = w_ref[...] * s_ref[...]
    out_ref[0:NUM_BINS, :] = -ws
    out_ref[NUM_BINS:2 * NUM_BINS, :] = jnp.zeros_like(ws)
    out_ref[2 * NUM_BINS:3 * NUM_BINS, :] = ws


def _sc_gather(idx_hbm, w3_hbm, out_hbm, idx_v, buf_v, w3_sh, gsem, wsem):
    wid = lax.axis_index("s") * NC + lax.axis_index("c")
    sid = lax.axis_index("s")
    chunks_per_w = idx_hbm.shape[1]
    base = wid * chunks_per_w * CHUNK

    @pl.when(sid == 0)
    def _():
        pltpu.sync_copy(w3_hbm, w3_sh)

    plsc.subcore_barrier()

    pltpu.sync_copy(idx_hbm.at[wid], idx_v)

    ngroups = chunks_per_w // KFIRE

    def out_slice(g):
        return out_hbm.at[pl.ds(base + g * CHUNK, CHUNK)]

    for j in range(KFIRE):
        pltpu.async_copy(buf_v.at[j], out_slice(j), wsem)

    def group(g0, carry):
        gathers = []
        for j in range(KFIRE):
            g = g0 * KFIRE + j
            pltpu.make_async_copy(buf_v.at[j], out_slice(g), wsem).wait()
            gathers.append(
                pltpu.async_copy(w3_sh.at[idx_v.at[g]], buf_v.at[j], gsem)
            )
        for j in range(KFIRE):
            g = g0 * KFIRE + j
            gathers[j].wait()
            pltpu.async_copy(buf_v.at[j], out_slice(g), wsem)
        return carry

    lax.fori_loop(0, ngroups, group, 0)

    for j in range(KFIRE):
        pltpu.make_async_copy(buf_v.at[j], out_slice(j), wsem).wait()


def kernel(number, W, scale):
    squeeze = number.ndim == 1
    if squeeze:
        number = number[None, :]
    B, N = number.shape
    M = B * N
    assert M % (NW * CHUNK) == 0
    chunks_per_w = M // (NW * CHUNK)

    rows_blk = max(8, min(B, (1 << 22) // (4 * N)))
    while B % rows_blk:
        rows_blk //= 2
    idx = pl.pallas_call(
        _idx_body,
        grid=(B // rows_blk,),
        in_specs=[pl.BlockSpec((rows_blk, N), lambda i: (i, 0))],
        out_specs=pl.BlockSpec((rows_blk, N), lambda i: (i, 0)),
        out_shape=jax.ShapeDtypeStruct((B, N), jnp.int32),
    )(number)

    w3 = pl.pallas_call(
        _table_body,
        out_shape=jax.ShapeDtypeStruct((3 * NUM_BINS, EMB_DIM), jnp.float32),
    )(W, scale.reshape(NUM_BINS, 1))

    idx3 = idx.reshape(NW, chunks_per_w, CHUNK)

    mesh = plsc.VectorSubcoreMesh(
        core_axis_name="c", subcore_axis_name="s", num_cores=NC, num_subcores=NS
    )
    out = pl.kernel(
        _sc_gather,
        out_type=jax.ShapeDtypeStruct((M, EMB_DIM), jnp.float32),
        mesh=mesh,
        scratch_types=[
            pltpu.VMEM((chunks_per_w, CHUNK), jnp.int32),
            pltpu.VMEM((KFIRE, CHUNK, EMB_DIM), jnp.float32),
            pltpu.VMEM_SHARED((3 * NUM_BINS, EMB_DIM), jnp.float32),
            pltpu.SemaphoreType.DMA,
            pltpu.SemaphoreType.DMA,
        ],
        compiler_params=pltpu.CompilerParams(use_tc_tiling_on_sc=False),
    )(idx3, w3)

    out = out.reshape(B, N, EMB_DIM)
    if squeeze:
        out = out[0]
    return out


# speedup vs baseline: 3.6175x; 1.0002x over previous
import functools
import jax
import jax.numpy as jnp
from jax import lax
from jax.experimental import pallas as pl
from jax.experimental.pallas import tpu as pltpu
from jax.experimental.pallas import tpu_sc as plsc

NUM_BINS = 128
EMB_DIM = 64
NC = 2
NS = 16
NW = NC * NS

CHUNK = 128
KFIRE = 8


def _idx_body(num_ref, idx_ref):
    x = num_ref[...]
    l = jnp.log2(jnp.abs(x) + 1e-10)
    t = (l + 10.0) * 6.4
    k = jnp.clip(jnp.ceil(t).astype(jnp.int32), 0, 129)
    bk = -10.0 + k.astype(jnp.float32) * 0.15625
    bkm1 = -10.0 + (k - 1).astype(jnp.float32) * 0.15625
    k = jnp.where((k <= 128) & (bk < l), k + 1, k)
    k = jnp.where((k >= 1) & (bkm1 >= l), k - 1, k)
    b = jnp.clip(k, 0, 127)
    sgn = (x > 0.0).astype(jnp.int32) - (x < 0.0).astype(jnp.int32)
    idx_ref[...] = b + 128 * (sgn + 1)


def _table_body(w_ref, s_ref, out_ref):
    ws = w_ref[...] * s_ref[...]
    out_ref[0:NUM_BINS, :] = -ws
    out_ref[NUM_BINS:2 * NUM_BINS, :] = jnp.zeros_like(ws)
    out_ref[2 * NUM_BINS:3 * NUM_BINS, :] = ws


def _sc_gather(idx_hbm, w3_hbm, out_hbm, idx_v, buf_v, w3_sh, gsem, wsem):
    wid = lax.axis_index("s") * NC + lax.axis_index("c")
    sid = lax.axis_index("s")
    chunks_per_w = idx_hbm.shape[1]
    n_chunks = out_hbm.shape[1] // CHUNK

    @pl.when(sid == 0)
    def _():
        pltpu.sync_copy(w3_hbm, w3_sh)

    plsc.subcore_barrier()

    pltpu.sync_copy(idx_hbm.at[wid], idx_v)

    ngroups = chunks_per_w // KFIRE

    rows_per_w = chunks_per_w // n_chunks

    def out_slice(g):
        b = wid * rows_per_w + g // n_chunks
        n0 = (g % n_chunks) * CHUNK
        return out_hbm.at[b, pl.ds(n0, CHUNK)]

    for j in range(KFIRE):
        pltpu.async_copy(buf_v.at[j], out_slice(j), wsem)

    def group(g0, carry):
        gathers = []
        for j in range(KFIRE):
            g = g0 * KFIRE + j
            pltpu.make_async_copy(buf_v.at[j], out_slice(g), wsem).wait()
            gathers.append(
                pltpu.async_copy(w3_sh.at[idx_v.at[g]], buf_v.at[j], gsem)
            )
        for j in range(KFIRE):
            g = g0 * KFIRE + j
            gathers[j].wait()
            pltpu.async_copy(buf_v.at[j], out_slice(g), wsem)
        return carry

    lax.fori_loop(0, ngroups, group, 0)

    for j in range(KFIRE):
        pltpu.make_async_copy(buf_v.at[j], out_slice(j), wsem).wait()


def kernel(number, W, scale):
    squeeze = number.ndim == 1
    if squeeze:
        number = number[None, :]
    B, N = number.shape
    M = B * N
    assert M % (NW * CHUNK) == 0
    chunks_per_w = M // (NW * CHUNK)

    rows_blk = max(8, min(B, (1 << 22) // (4 * N)))
    while B % rows_blk:
        rows_blk //= 2
    idx = pl.pallas_call(
        _idx_body,
        grid=(B // rows_blk,),
        in_specs=[pl.BlockSpec((rows_blk, N), lambda i: (i, 0))],
        out_specs=pl.BlockSpec((rows_blk, N), lambda i: (i, 0)),
        out_shape=jax.ShapeDtypeStruct((B, N), jnp.int32),
    )(number)

    w3 = pl.pallas_call(
        _table_body,
        out_shape=jax.ShapeDtypeStruct((3 * NUM_BINS, EMB_DIM), jnp.float32),
    )(W, scale.reshape(NUM_BINS, 1))

    idx3 = idx.reshape(NW, chunks_per_w, CHUNK)

    mesh = plsc.VectorSubcoreMesh(
        core_axis_name="c", subcore_axis_name="s", num_cores=NC, num_subcores=NS
    )
    out = pl.kernel(
        _sc_gather,
        out_type=jax.ShapeDtypeStruct((B, N, EMB_DIM), jnp.float32),
        mesh=mesh,
        scratch_types=[
            pltpu.VMEM((chunks_per_w, CHUNK), jnp.int32),
            pltpu.VMEM((KFIRE, CHUNK, EMB_DIM), jnp.float32),
            pltpu.VMEM_SHARED((3 * NUM_BINS, EMB_DIM), jnp.float32),
            pltpu.SemaphoreType.DMA,
            pltpu.SemaphoreType.DMA,
        ],
        compiler_params=pltpu.CompilerParams(use_tc_tiling_on_sc=False),
    )(idx3, w3)

    if squeeze:
        out = out[0]
    return out
